# baseline (speedup 1.0000x reference)
import functools

import jax
import jax.numpy as jnp
from jax import lax
from jax.experimental import pallas as pl
from jax.experimental.pallas import tpu as pltpu
from jax.experimental.pallas import tpu_sc as plsc

BATCH = 16384
D = 64
LANES = 16
N_CORES = 2
N_SUBCORES = 16
N_WORKERS = N_CORES * N_SUBCORES
B_PER_W = BATCH // N_WORKERS
CHUNK = 128
N_CHUNKS = B_PER_W // CHUNK
N_GROUPS = B_PER_W // LANES


def _mf_body(user_hbm, item_hbm, uf_hbm, if_hbm, out_hbm,
             idx_u, idx_i, rows_u, rows_i, out_v, sem):
    wid = lax.axis_index("s") * N_CORES + lax.axis_index("c")
    base = wid * B_PER_W

    for j in range(N_CHUNKS):
        pltpu.sync_copy(user_hbm.at[pl.ds(base + j * CHUNK, CHUNK)], idx_u.at[j])
        pltpu.sync_copy(item_hbm.at[pl.ds(base + j * CHUNK, CHUNK)], idx_i.at[j])

    copies = []
    for j in range(N_CHUNKS):
        copies.append(pltpu.async_copy(
            uf_hbm.at[idx_u.at[j]], rows_u.at[pl.ds(j * CHUNK, CHUNK)], sem))
        copies.append(pltpu.async_copy(
            if_hbm.at[idx_i.at[j]], rows_i.at[pl.ds(j * CHUNK, CHUNK)], sem))
    for c in copies:
        c.wait()

    row_iota = lax.iota(jnp.int32, LANES)
    cols = [jnp.full((LANES,), f, jnp.int32) for f in range(D)]

    def group(g, _):
        rows = g * LANES + row_iota
        accs = [jnp.zeros((LANES,), jnp.float32) for _ in range(4)]
        for f in range(D):
            u16 = plsc.load_gather(rows_u, [rows, cols[f]])
            i16 = plsc.load_gather(rows_i, [rows, cols[f]])
            accs[f % 4] = accs[f % 4] + u16 * i16
        out_v[pl.ds(g * LANES, LANES)] = (accs[0] + accs[1]) + (accs[2] + accs[3])
        return _

    lax.fori_loop(0, N_GROUPS, group, None)
    pltpu.sync_copy(out_v, out_hbm.at[pl.ds(base, B_PER_W)])


@functools.partial(
    pl.kernel,
    out_type=jax.ShapeDtypeStruct((BATCH,), jnp.float32),
    mesh=plsc.VectorSubcoreMesh(core_axis_name="c", subcore_axis_name="s"),
    compiler_params=pltpu.CompilerParams(
        use_tc_tiling_on_sc=False, needs_layout_passes=False),
    scratch_types=[
        pltpu.VMEM((N_CHUNKS, CHUNK), jnp.int32),
        pltpu.VMEM((N_CHUNKS, CHUNK), jnp.int32),
        pltpu.VMEM((B_PER_W, D), jnp.float32),
        pltpu.VMEM((B_PER_W, D), jnp.float32),
        pltpu.VMEM((B_PER_W,), jnp.float32),
        pltpu.SemaphoreType.DMA,
    ],
)
def _mf_kernel(user_hbm, item_hbm, uf_hbm, if_hbm, out_hbm,
               idx_u, idx_i, rows_u, rows_i, out_v, sem):
    _mf_body(user_hbm, item_hbm, uf_hbm, if_hbm, out_hbm,
             idx_u, idx_i, rows_u, rows_i, out_v, sem)


def kernel(user, item, user_factors, item_factors):
    return _mf_kernel(user.astype(jnp.int32), item.astype(jnp.int32),
                      user_factors, item_factors)


# speedup vs baseline: 1.0246x; 1.0246x over previous
import functools

import jax
import jax.numpy as jnp
from jax import lax
from jax.experimental import pallas as pl
from jax.experimental.pallas import tpu as pltpu
from jax.experimental.pallas import tpu_sc as plsc

BATCH = 16384
D = 64
W = 128
LANES = 16
N_CORES = 2
N_SUBCORES = 16
N_WORKERS = N_CORES * N_SUBCORES
B_PER_W = BATCH // N_WORKERS
CHUNK = 256
N_CHUNKS = B_PER_W // CHUNK
N_GROUPS = CHUNK // LANES


@functools.partial(
    pl.kernel,
    out_type=jax.ShapeDtypeStruct((BATCH,), jnp.float32),
    mesh=plsc.VectorSubcoreMesh(core_axis_name="c", subcore_axis_name="s"),
    compiler_params=pltpu.CompilerParams(needs_layout_passes=False),
    scratch_types=[
        pltpu.VMEM((B_PER_W,), jnp.int32),
        pltpu.VMEM((B_PER_W,), jnp.int32),
        pltpu.VMEM((B_PER_W,), jnp.int32),
        pltpu.VMEM((B_PER_W,), jnp.int32),
        pltpu.VMEM((CHUNK, W), jnp.float32),
        pltpu.VMEM((CHUNK, W), jnp.float32),
        pltpu.VMEM((B_PER_W,), jnp.float32),
        pltpu.SemaphoreType.DMA,
        pltpu.SemaphoreType.DMA,
    ],
)
def _mf_kernel(ru_hbm, ri_hbm, ou_hbm, oi_hbm, tu_hbm, ti_hbm, out_hbm,
               idx_u, idx_i, off_u, off_i, rows_u, rows_i, out_v,
               sem_u, sem_i):
    wid = lax.axis_index("s") * N_CORES + lax.axis_index("c")
    base = wid * B_PER_W

    pltpu.sync_copy(ru_hbm.at[pl.ds(base, B_PER_W)], idx_u)
    pltpu.sync_copy(ri_hbm.at[pl.ds(base, B_PER_W)], idx_i)
    pltpu.sync_copy(ou_hbm.at[pl.ds(base, B_PER_W)], off_u)
    pltpu.sync_copy(oi_hbm.at[pl.ds(base, B_PER_W)], off_i)

    lane = lax.iota(jnp.int32, LANES)
    last_lane = lane == (LANES - 1)

    for c in range(N_CHUNKS):
        c0 = c * CHUNK
        cp_u = pltpu.async_copy(tu_hbm.at[idx_u.at[pl.ds(c0, CHUNK)]],
                                rows_u, sem_u)
        cp_i = pltpu.async_copy(ti_hbm.at[idx_i.at[pl.ds(c0, CHUNK)]],
                                rows_i, sem_i)
        cp_u.wait()
        cp_i.wait()

        def group(g, _, c0=c0):
            b0 = c0 + g * LANES
            ou_vec = off_u[pl.ds(b0, LANES)]
            oi_vec = off_i[pl.ds(b0, LANES)]
            for j in range(LANES):
                row = g * LANES + j
                pu = ou_vec[j] == 0
                pi = oi_vec[j] == 0
                acc = None
                for k in range(D // LANES):
                    u = jnp.where(pu, rows_u[row, pl.ds(k * LANES, LANES)],
                                  rows_u[row, pl.ds(D + k * LANES, LANES)])
                    v = jnp.where(pi, rows_i[row, pl.ds(k * LANES, LANES)],
                                  rows_i[row, pl.ds(D + k * LANES, LANES)])
                    acc = u * v if acc is None else acc + u * v
                total = plsc.cumsum(acc)
                plsc.store_scatter(out_v,
                                   [jnp.full((LANES,), b0 + j, jnp.int32)],
                                   total, mask=last_lane)
            return _

        lax.fori_loop(0, N_GROUPS, group, None)

    pltpu.sync_copy(out_v, out_hbm.at[pl.ds(base, B_PER_W)])


def kernel(user, item, user_factors, item_factors):
    user = user.astype(jnp.int32)
    item = item.astype(jnp.int32)
    tu = user_factors.reshape(user_factors.shape[0] // 2, 2 * D)
    ti = item_factors.reshape(item_factors.shape[0] // 2, 2 * D)
    return _mf_kernel(user >> 1, item >> 1, user & 1, item & 1, tu, ti)


# speedup vs baseline: 1.1358x; 1.1086x over previous
import functools

import jax
import jax.numpy as jnp
from jax import lax
from jax.experimental import pallas as pl
from jax.experimental.pallas import tpu as pltpu
from jax.experimental.pallas import tpu_sc as plsc

BATCH = 16384
D = 64
W = 128
LANES = 16
N_CORES = 2
N_SUBCORES = 16
N_WORKERS = N_CORES * N_SUBCORES
B_PER_W = BATCH // N_WORKERS
CHUNK = 256
N_CHUNKS = B_PER_W // CHUNK
N_GROUPS = CHUNK // LANES


@functools.partial(
    pl.kernel,
    out_type=jax.ShapeDtypeStruct((BATCH,), jnp.float32),
    mesh=plsc.VectorSubcoreMesh(core_axis_name="c", subcore_axis_name="s"),
    compiler_params=pltpu.CompilerParams(needs_layout_passes=False),
    scratch_types=[
        pltpu.VMEM((B_PER_W,), jnp.int32),
        pltpu.VMEM((B_PER_W,), jnp.int32),
        pltpu.VMEM((CHUNK, W), jnp.float32),
        pltpu.VMEM((CHUNK, W), jnp.float32),
        pltpu.VMEM((B_PER_W,), jnp.float32),
        pltpu.SemaphoreType.DMA,
        pltpu.SemaphoreType.DMA,
    ],
)
def _mf_kernel(user_hbm, item_hbm, tu_hbm, ti_hbm, out_hbm,
               idx_u, idx_i, rows_u, rows_i, out_v, sem_u, sem_i):
    wid = lax.axis_index("s") * N_CORES + lax.axis_index("c")
    base = wid * B_PER_W

    pltpu.sync_copy(user_hbm.at[pl.ds(base, B_PER_W)], idx_u)
    pltpu.sync_copy(item_hbm.at[pl.ds(base, B_PER_W)], idx_i)

    lane = lax.iota(jnp.int32, LANES)
    last_lane = lane == (LANES - 1)

    for c in range(N_CHUNKS):
        c0 = c * CHUNK
        cp_u = pltpu.async_copy(tu_hbm.at[idx_u.at[pl.ds(c0, CHUNK)]],
                                rows_u, sem_u)
        cp_i = pltpu.async_copy(ti_hbm.at[idx_i.at[pl.ds(c0, CHUNK)]],
                                rows_i, sem_i)
        cp_u.wait()
        cp_i.wait()

        def group(g, _, c0=c0):
            b0 = c0 + g * LANES
            for j in range(LANES):
                row = g * LANES + j
                acc = None
                for k in range(D // LANES):
                    u = rows_u[row, pl.ds(k * LANES, LANES)]
                    v = rows_i[row, pl.ds(k * LANES, LANES)]
                    acc = u * v if acc is None else acc + u * v
                total = plsc.cumsum(acc)
                plsc.store_scatter(out_v,
                                   [jnp.full((LANES,), b0 + j, jnp.int32)],
                                   total, mask=last_lane)
            return _

        lax.fori_loop(0, N_GROUPS, group, None)

    pltpu.sync_copy(out_v, out_hbm.at[pl.ds(base, B_PER_W)])


def kernel(user, item, user_factors, item_factors):
    user = user.astype(jnp.int32)
    item = item.astype(jnp.int32)
    tu = jnp.pad(user_factors, ((0, 0), (0, W - D)))
    ti = jnp.pad(item_factors, ((0, 0), (0, W - D)))
    return _mf_kernel(user, item, tu, ti)


# speedup vs baseline: 2.5069x; 2.2071x over previous
import functools

import jax
import jax.numpy as jnp
from jax import lax
from jax.experimental import pallas as pl
from jax.experimental.pallas import tpu as pltpu
from jax.experimental.pallas import tpu_sc as plsc

BATCH = 16384
D = 64
SUB = 8
LANES = 16
N_CORES = 2
N_SUBCORES = 16
N_WORKERS = N_CORES * N_SUBCORES
B_PER_W = BATCH // N_WORKERS
CHUNK = 256
N_CHUNKS = B_PER_W // CHUNK
N_GROUPS = CHUNK // LANES


@functools.partial(
    pl.kernel,
    out_type=jax.ShapeDtypeStruct((BATCH,), jnp.float32),
    mesh=plsc.VectorSubcoreMesh(core_axis_name="c", subcore_axis_name="s"),
    compiler_params=pltpu.CompilerParams(needs_layout_passes=False),
    scratch_types=[
        pltpu.VMEM((B_PER_W,), jnp.int32),
        pltpu.VMEM((B_PER_W,), jnp.int32),
        pltpu.VMEM((CHUNK, D), jnp.float32),
        pltpu.VMEM((CHUNK, D), jnp.float32),
        pltpu.VMEM((B_PER_W,), jnp.float32),
        pltpu.SemaphoreType.DMA,
        pltpu.SemaphoreType.DMA,
    ],
)
def _mf_kernel(user_hbm, item_hbm, tu_hbm, ti_hbm, out_hbm,
               idx_u, idx_i, rows_u, rows_i, out_v, sem_u, sem_i):
    wid = lax.axis_index("s") * N_CORES + lax.axis_index("c")
    base = wid * B_PER_W

    pltpu.sync_copy(user_hbm.at[pl.ds(base, B_PER_W)], idx_u)
    pltpu.sync_copy(item_hbm.at[pl.ds(base, B_PER_W)], idx_i)

    lane = lax.iota(jnp.int32, LANES)
    last_lane = lane == (LANES - 1)

    for c in range(N_CHUNKS):
        c0 = c * CHUNK

        def fire(g, _, c0=c0):
            b0 = c0 + g * LANES
            vu = idx_u[pl.ds(b0, LANES)]
            vi = idx_i[pl.ds(b0, LANES)]
            for j in range(LANES):
                row = g * LANES + j
                ru = vu[j]
                pltpu.async_copy(
                    tu_hbm.at[ru >> 3, pl.ds(ru & 7, 1), :],
                    rows_u.at[pl.ds(row, 1), :], sem_u)
                ri = vi[j]
                pltpu.async_copy(
                    ti_hbm.at[ri >> 3, pl.ds(ri & 7, 1), :],
                    rows_i.at[pl.ds(row, 1), :], sem_i)
            return _

        lax.fori_loop(0, N_GROUPS, fire, None)
        for start in range(0, CHUNK, SUB):
            pltpu.make_async_copy(tu_hbm.at[0],
                                  rows_u.at[pl.ds(start, SUB), :],
                                  sem_u).wait()
        for start in range(0, CHUNK, SUB):
            pltpu.make_async_copy(ti_hbm.at[0],
                                  rows_i.at[pl.ds(start, SUB), :],
                                  sem_i).wait()

        def group(g, _, c0=c0):
            b0 = c0 + g * LANES
            for j in range(LANES):
                row = g * LANES + j
                acc = None
                for k in range(D // LANES):
                    u = rows_u[row, pl.ds(k * LANES, LANES)]
                    v = rows_i[row, pl.ds(k * LANES, LANES)]
                    acc = u * v if acc is None else acc + u * v
                total = plsc.cumsum(acc)
                plsc.store_scatter(out_v,
                                   [jnp.full((LANES,), b0 + j, jnp.int32)],
                                   total, mask=last_lane)
            return _

        lax.fori_loop(0, N_GROUPS, group, None)

    pltpu.sync_copy(out_v, out_hbm.at[pl.ds(base, B_PER_W)])


def kernel(user, item, user_factors, item_factors):
    user = user.astype(jnp.int32)
    item = item.astype(jnp.int32)
    tu = user_factors.reshape(user_factors.shape[0] // SUB, SUB, D)
    ti = item_factors.reshape(item_factors.shape[0] // SUB, SUB, D)
    return _mf_kernel(user, item, tu, ti)
